# baseline (speedup 1.0000x reference)
import functools

import jax
import jax.numpy as jnp
from jax import lax
from jax.experimental import pallas as pl
from jax.experimental.pallas import tpu as pltpu
from jax.experimental.pallas import tpu_sc as plsc

NUM_CORES = 2
NUM_SUBCORES = 16
NW = NUM_CORES * NUM_SUBCORES
LANES = 16


@functools.lru_cache(maxsize=None)
def _make_embed(batch, seq, vocab, maxlen, embed, interpret=False):
    assert batch % NW == 0
    assert embed % LANES == 0
    assert seq % 8 == 0
    assert seq <= maxlen
    seq_per_w = batch // NW
    assert seq_per_w % 2 == 0
    groups = embed // LANES
    n_sub = 5 if seq % 40 == 0 else 1
    sub_rows = seq // n_sub
    mesh = plsc.VectorSubcoreMesh(
        core_axis_name="c", subcore_axis_name="s",
        num_cores=NUM_CORES, num_subcores=NUM_SUBCORES)

    @functools.partial(
        pl.kernel,
        out_type=jax.ShapeDtypeStruct((batch, seq, embed), jnp.float32),
        mesh=mesh,
        scratch_types=[
            pltpu.VMEM((seq_per_w, seq), jnp.int32),
            pltpu.VMEM((seq, embed), jnp.float32),
            pltpu.VMEM((seq, embed), jnp.float32),
            pltpu.VMEM((maxlen, embed), jnp.float32),
            pltpu.SemaphoreType.DMA,
            pltpu.SemaphoreType.DMA,
            pltpu.SemaphoreType.DMA,
            pltpu.SemaphoreType.DMA,
        ],
        compiler_params=pltpu.CompilerParams(use_tc_tiling_on_sc=False),
        interpret=interpret,
    )
    def embed_kernel(x_hbm, tok_hbm, pos_hbm, out_hbm, idx_v, rows0, rows1,
                     pos_v, sg0, sg1, so0, so1):
        wid = lax.axis_index("s") * NUM_CORES + lax.axis_index("c")
        base = wid * seq_per_w
        rows = (rows0, rows1)
        sg = (sg0, sg1)
        so = (so0, so1)

        pltpu.sync_copy(pos_hbm.at[pl.ds(0, seq)], pos_v)
        pltpu.sync_copy(x_hbm.at[pl.ds(base, seq_per_w)], idx_v)

        def gather(r, b):
            for i in range(n_sub):
                pltpu.async_copy(
                    tok_hbm.at[idx_v.at[r].at[pl.ds(i * sub_rows, sub_rows)]],
                    rows[b].at[pl.ds(i * sub_rows, sub_rows)], sg[b])

        def wait_gather(b):
            for i in range(n_sub):
                pltpu.make_async_copy(
                    tok_hbm.at[idx_v.at[0].at[pl.ds(0, sub_rows)]],
                    rows[b].at[pl.ds(i * sub_rows, sub_rows)], sg[b]).wait()

        def put_out(r, b):
            pltpu.async_copy(rows[b], out_hbm.at[base + r], so[b])

        def wait_out(b):
            pltpu.make_async_copy(rows[b], out_hbm.at[base], so[b]).wait()

        gather(0, 0)

        def step(r, b):
            wait_gather(b)

            @pl.when(r + 1 < seq_per_w)
            def _():
                @pl.when(r >= 1)
                def _():
                    wait_out(1 - b)

                gather(r + 1, 1 - b)

            def add_body(j, carry):
                for k in range(groups):
                    sl = pl.ds(k * LANES, LANES)
                    rows[b][j, sl] = rows[b][j, sl] + pos_v[j, sl]
                return carry

            lax.fori_loop(0, seq, add_body, 0, unroll=2)
            put_out(r, b)
            return b

        def pair(r0, carry):
            step(r0 * 2, 0)
            step(r0 * 2 + 1, 1)
            return carry

        lax.fori_loop(0, seq_per_w // 2, pair, 0)
        wait_out(0)
        wait_out(1)

    return embed_kernel


def kernel(x, token_table, pos_table):
    batch, seq = x.shape
    vocab, embed = token_table.shape
    maxlen = pos_table.shape[0]
    fn = _make_embed(batch, seq, vocab, maxlen, embed)
    return fn(x.astype(jnp.int32), token_table, pos_table)


# speedup vs baseline: 1.0598x; 1.0598x over previous
import functools

import jax
import jax.numpy as jnp
from jax import lax
from jax.experimental import pallas as pl
from jax.experimental.pallas import tpu as pltpu
from jax.experimental.pallas import tpu_sc as plsc

NUM_CORES = 2
NUM_SUBCORES = 16
NW = NUM_CORES * NUM_SUBCORES
LANES = 16


@functools.lru_cache(maxsize=None)
def _make_embed(batch, seq, vocab2, maxlen, embed, interpret=False):
    assert batch % NW == 0
    assert embed % LANES == 0
    assert seq % 16 == 0 or seq % 2 == 0
    assert seq <= maxlen
    seq_per_w = batch // NW
    groups = embed // LANES
    c0 = (seq // 2) // LANES * LANES
    chunk_offs = (0, c0)
    chunk_sizes = (c0, seq - c0)
    assert all(s >= LANES and s % 8 == 0 for s in chunk_sizes)
    assert c0 % LANES == 0
    buf_rows = max(chunk_sizes)
    n_chunks = 2 * seq_per_w

    def sub_bounds(size):
        bounds, o = [], 0
        while o < size:
            n = min(40, size - o)
            bounds.append((o, n))
            o += n
        return tuple(bounds)
    mesh = plsc.VectorSubcoreMesh(
        core_axis_name="c", subcore_axis_name="s",
        num_cores=NUM_CORES, num_subcores=NUM_SUBCORES)

    @functools.partial(
        pl.kernel,
        out_type=jax.ShapeDtypeStruct((batch, seq, embed), jnp.float32),
        mesh=mesh,
        scratch_types=[
            pltpu.VMEM((seq_per_w, seq), jnp.int32),
            pltpu.VMEM((buf_rows,), jnp.int32),
            pltpu.VMEM((buf_rows,), jnp.int32),
            pltpu.VMEM((buf_rows, 2 * embed), jnp.float32),
            pltpu.VMEM((buf_rows, 2 * embed), jnp.float32),
            pltpu.VMEM((buf_rows, embed), jnp.float32),
            pltpu.VMEM((buf_rows, embed), jnp.float32),
            pltpu.VMEM((seq, embed), jnp.float32),
            pltpu.SemaphoreType.DMA,
            pltpu.SemaphoreType.DMA,
            pltpu.SemaphoreType.DMA,
            pltpu.SemaphoreType.DMA,
        ],
        compiler_params=pltpu.CompilerParams(use_tc_tiling_on_sc=True),
        interpret=interpret,
    )
    def embed_kernel(x_hbm, tok2_hbm, pos_hbm, out_hbm, idx_v, ridx0, ridx1,
                     pair0, pair1, outb0, outb1, pos_v, sg0, sg1, so0, so1):
        wid = lax.axis_index("s") * NUM_CORES + lax.axis_index("c")
        base = wid * seq_per_w
        ridx = (ridx0, ridx1)
        pair = (pair0, pair1)
        outb = (outb0, outb1)
        sg = (sg0, sg1)
        so = (so0, so1)

        pltpu.sync_copy(pos_hbm.at[pl.ds(0, seq)], pos_v)
        pltpu.sync_copy(x_hbm.at[pl.ds(base, seq_per_w)], idx_v)

        def group_offs(size):
            offs = [g * LANES for g in range(size // LANES)]
            if size % LANES:
                offs.append(size - LANES)
            return offs

        def gather(r, half, b):
            off = chunk_offs[half]
            size = chunk_sizes[half]

            def shift_at(jb):
                ridx[b][pl.ds(jb, LANES)] = lax.shift_right_logical(
                    idx_v.at[r][pl.ds(off + jb, LANES)],
                    jnp.full((LANES,), 1, jnp.int32))

            def shift_body(i, carry):
                shift_at(i * LANES)
                return carry

            lax.fori_loop(0, size // LANES, shift_body, 0)
            if size % LANES:
                shift_at(size - LANES)
            for (o, n) in sub_bounds(size):
                pltpu.async_copy(
                    tok2_hbm.at[ridx[b].at[pl.ds(o, n)]],
                    pair[b].at[pl.ds(o, n)], sg[b])

        def wait_gather(half, b):
            for (o, n) in sub_bounds(chunk_sizes[half]):
                pltpu.make_async_copy(
                    tok2_hbm.at[ridx[b].at[pl.ds(0, n)]],
                    pair[b].at[pl.ds(o, n)], sg[b]).wait()

        def put_out(r, half, b):
            off = chunk_offs[half]
            size = chunk_sizes[half]
            pltpu.async_copy(
                outb[b].at[pl.ds(0, size)],
                out_hbm.at[base + r].at[pl.ds(off, size)], so[b])

        def wait_out(half, b):
            size = chunk_sizes[half]
            pltpu.make_async_copy(
                outb[b].at[pl.ds(0, size)],
                out_hbm.at[base].at[pl.ds(0, size)], so[b]).wait()

        gather(0, 0, 0)

        def step(r, half):
            b = half
            wait_gather(half, b)

            if half == 0:
                @pl.when(r >= 1)
                def _():
                    wait_out(1, 1)

                gather(r, 1, 1)
            else:
                @pl.when(r < seq_per_w - 1)
                def _():
                    wait_out(0, 0)
                    gather(r + 1, 0, 0)

            off = chunk_offs[half]
            size = chunk_sizes[half]

            def add_at(jb):
                idxg = idx_v.at[r][pl.ds(off + jb, LANES)]
                oddv = idxg & 1
                for l in range(LANES):
                    j = jb + l
                    odd = oddv[l] == 1
                    for k in range(groups):
                        sl = pl.ds(k * LANES, LANES)
                        lo = pair[b][j, sl]
                        hi = pair[b][j, pl.ds(embed + k * LANES, LANES)]
                        outb[b][j, sl] = (
                            jnp.where(odd, hi, lo) + pos_v[off + j, sl])

            def add_body(g, carry):
                add_at(g * LANES)
                return carry

            lax.fori_loop(0, size // LANES, add_body, 0)
            if size % LANES:
                add_at(size - LANES)
            put_out(r, half, b)

        def pair_step(r, carry):
            step(r, 0)
            step(r, 1)
            return carry

        lax.fori_loop(0, seq_per_w, pair_step, 0)
        wait_out(0, 0)
        wait_out(1, 1)

    return embed_kernel


def kernel(x, token_table, pos_table):
    batch, seq = x.shape
    vocab, embed = token_table.shape
    maxlen = pos_table.shape[0]
    assert vocab % 2 == 0
    tok2 = token_table.reshape(vocab // 2, 2 * embed)
    fn = _make_embed(batch, seq, vocab // 2, maxlen, embed)
    return fn(x.astype(jnp.int32), tok2, pos_table)
